# baseline (speedup 1.0000x reference)
import functools

import jax
import jax.numpy as jnp
from jax import lax
from jax.experimental import pallas as pl
from jax.experimental.pallas import tpu as pltpu
from jax.experimental.pallas import tpu_sc as plsc

N = 10000
E = 320000
D = 128
DE = 16

_CHUNK = 128
_NCHUNK = E // _CHUNK
_NW = 32
_MAXTRIP = (_NCHUNK + _NW - 1) // _NW
_ROWS_PER_TILE = N // 16


def _pq_tc(x, w1a, w1b, b1):
  def body(x_ref, wa_ref, wb_ref, b_ref, p_ref, q_ref):
    xv = x_ref[...]
    p_ref[...] = jnp.dot(xv, wa_ref[...],
                         preferred_element_type=jnp.float32) + b_ref[...]
    q_ref[...] = jnp.dot(xv, wb_ref[...], preferred_element_type=jnp.float32)

  bn = 2000
  grid = N // bn
  return pl.pallas_call(
      body,
      grid=(grid,),
      in_specs=[
          pl.BlockSpec((bn, D), lambda i: (i, 0)),
          pl.BlockSpec((D, DE), lambda i: (0, 0)),
          pl.BlockSpec((D, DE), lambda i: (0, 0)),
          pl.BlockSpec((1, DE), lambda i: (0, 0)),
      ],
      out_specs=[
          pl.BlockSpec((bn, DE), lambda i: (i, 0)),
          pl.BlockSpec((bn, DE), lambda i: (i, 0)),
      ],
      out_shape=[
          jax.ShapeDtypeStruct((N, DE), jnp.float32),
          jax.ShapeDtypeStruct((N, DE), jnp.float32),
      ],
  )(x, w1a, w1b, b1)


_KK = 13
_NBATCH = 6


def _gather_sc(t, cidx):
  mesh = plsc.VectorSubcoreMesh(core_axis_name="c", subcore_axis_name="s")

  @functools.partial(
      pl.kernel,
      mesh=mesh,
      compiler_params=pltpu.CompilerParams(use_tc_tiling_on_sc=False),
      out_type=jax.ShapeDtypeStruct((_NCHUNK, DE, _CHUNK), jnp.float32),
      scratch_types=[
          pltpu.VMEM((_KK, 2 * _CHUNK), jnp.int32),
          pltpu.VMEM((_KK, _CHUNK, DE), jnp.float32),
          pltpu.VMEM((_KK, _CHUNK, DE), jnp.float32),
          pltpu.VMEM((_KK, DE, _CHUNK), jnp.float32),
          pltpu.SemaphoreType.DMA,
          pltpu.SemaphoreType.DMA,
          pltpu.SemaphoreType.DMA,
      ],
      name="gather_sc",
  )
  def k(t_hbm, cidx_hbm, g_hbm, idxb, pb, qb, gb, s_i, s_g, s_o):
    c = lax.axis_index("c")
    s = lax.axis_index("s")
    wid = s * 2 + c

    def repack(kk):
      def rows(r8, carry):
        for m in range(8):
          v = pb[kk, r8 * 8 + m, :] + qb[kk, r8 * 8 + m, :]
          gb[kk, r8, m * DE:(m + 1) * DE] = v
        return carry

      lax.fori_loop(0, DE, rows, 0)

    def batch(g, carry):
      cps = []
      for kk in range(_KK):
        cid = wid + (g * _KK + kk) * _NW
        cps.append(pltpu.async_copy(cidx_hbm.at[cid], idxb.at[kk], s_i))
      for cp in cps:
        cp.wait()
      cps = []
      for kk in range(_KK):
        cps.append(pltpu.async_copy(
            t_hbm.at[idxb.at[kk, pl.ds(0, _CHUNK)]], pb.at[kk], s_g))
        cps.append(pltpu.async_copy(
            t_hbm.at[idxb.at[kk, pl.ds(_CHUNK, _CHUNK)]], qb.at[kk], s_g))
      for cp in cps:
        cp.wait()
      for kk in range(_KK):
        repack(kk)
      cps = []
      for kk in range(_KK):
        cid = wid + (g * _KK + kk) * _NW
        cps.append(pltpu.async_copy(gb.at[kk], g_hbm.at[cid], s_o))
      for cp in cps:
        cp.wait()
      return carry

    lax.fori_loop(0, _NBATCH, batch, 0)

    cid = _NBATCH * _KK * _NW + wid

    @pl.when(cid < _NCHUNK)
    def _():
      pltpu.sync_copy(cidx_hbm.at[cid], idxb.at[0])
      cp1 = pltpu.async_copy(
          t_hbm.at[idxb.at[0, pl.ds(0, _CHUNK)]], pb.at[0], s_g)
      cp2 = pltpu.async_copy(
          t_hbm.at[idxb.at[0, pl.ds(_CHUNK, _CHUNK)]], qb.at[0], s_g)
      cp1.wait()
      cp2.wait()
      repack(0)
      pltpu.sync_copy(gb.at[0], g_hbm.at[cid])

  return k(t, cidx)


def _edge_mlp_tc(g3, ea, w1bd, w2bd, w3bd, b2t, b3t, egt, ebtt, jbd):
  bn = 1600
  rows = E // 8

  def body(g_ref, ea_ref, w1_ref, w2_ref, w3_ref, b2_ref, b3_ref,
           eg_ref, ebt_ref, jm_ref, out_ref, ea2_ref):
    eav = jnp.concatenate(
        [ea_ref[pl.Slice(m, bn, 8), :] for m in range(8)], axis=1)
    gv = g_ref[...].reshape(bn, _CHUNK)
    h1 = gv + jnp.dot(eav, w1_ref[...], preferred_element_type=jnp.float32)
    h1 = jax.nn.silu(h1)
    h2 = jax.nn.silu(jnp.dot(h1, w2_ref[...],
                             preferred_element_type=jnp.float32) + b2_ref[...])
    h3 = jnp.dot(h2, w3_ref[...],
                 preferred_element_type=jnp.float32) + b3_ref[...]
    jm = jm_ref[...]
    m_ = jnp.dot(h3, jm, preferred_element_type=jnp.float32)
    msq = jnp.dot(h3 * h3, jm, preferred_element_type=jnp.float32)
    var = msq - m_ * m_
    ln = (h3 - m_) * lax.rsqrt(var + 1e-5) * eg_ref[...] + ebt_ref[...]
    e2 = eav + ln
    out_ref[...] = e2
    for m in range(8):
      ea2_ref[pl.Slice(m, bn, 8), :] = e2[:, m * DE:(m + 1) * DE]

  grid = rows // bn
  wspec = pl.BlockSpec((D, D), lambda i: (0, 0))
  bspec = pl.BlockSpec((1, D), lambda i: (0, 0))
  return pl.pallas_call(
      body,
      grid=(grid,),
      in_specs=[
          pl.BlockSpec((bn // DE, DE, _CHUNK), lambda i: (i, 0, 0)),
          pl.BlockSpec((bn * 8, DE), lambda i: (i, 0)),
          wspec, wspec, wspec, bspec, bspec, bspec, bspec, wspec,
      ],
      out_specs=[
          pl.BlockSpec((bn, D), lambda i: (i, 0)),
          pl.BlockSpec((bn * 8, DE), lambda i: (i, 0)),
      ],
      out_shape=[
          jax.ShapeDtypeStruct((rows, D), jnp.float32),
          jax.ShapeDtypeStruct((E, DE), jnp.float32),
      ],
  )(g3, ea, w1bd, w2bd, w3bd, b2t, b3t, egt, ebtt, jbd)


def _scatter_sc(e2, j2d):
  mesh = plsc.VectorSubcoreMesh(core_axis_name="c", subcore_axis_name="s")

  @functools.partial(
      pl.kernel,
      mesh=mesh,
      compiler_params=pltpu.CompilerParams(use_tc_tiling_on_sc=False),
      out_type=jax.ShapeDtypeStruct((2 * N, DE), jnp.float32),
      name="scatter_sc",
      scratch_types=[
          pltpu.VMEM((_KK, _CHUNK), jnp.int32),
          pltpu.VMEM((_KK, _CHUNK, DE), jnp.float32),
          pltpu.VMEM((_ROWS_PER_TILE, DE), jnp.float32),
          pltpu.VMEM_SHARED((N, DE), jnp.float32),
          pltpu.SemaphoreType.DMA,
          pltpu.SemaphoreType.DMA,
      ],
  )
  def k(e2_hbm, j_hbm, out_hbm, jb, eb, wb, agg_sh, s_i, s_a):
    c = lax.axis_index("c")
    s = lax.axis_index("s")
    wid = s * 2 + c

    def zbody(r, carry):
      wb[r, :] = jnp.zeros((DE,), jnp.float32)
      return carry

    lax.fori_loop(0, _ROWS_PER_TILE, zbody, 0)
    pltpu.sync_copy(wb, agg_sh.at[pl.ds(s * _ROWS_PER_TILE, _ROWS_PER_TILE)])
    plsc.subcore_barrier()

    def batch(g, carry):
      cps = []
      for kk in range(_KK):
        cid = wid + (g * _KK + kk) * _NW
        cps.append(pltpu.async_copy(j_hbm.at[cid], jb.at[kk], s_i))
        cps.append(pltpu.async_copy(e2_hbm.at[cid], eb.at[kk], s_i))
      for cp in cps:
        cp.wait()
      cps = []
      for kk in range(_KK):
        cps.append(pltpu.async_copy(
            eb.at[kk], agg_sh.at[jb.at[kk]], s_a, add=True))
      for cp in cps:
        cp.wait()
      return carry

    lax.fori_loop(0, _NBATCH, batch, 0)

    cid = _NBATCH * _KK * _NW + wid

    @pl.when(cid < _NCHUNK)
    def _():
      pltpu.sync_copy(j_hbm.at[cid], jb.at[0])
      pltpu.sync_copy(e2_hbm.at[cid], eb.at[0])
      pltpu.sync_copy(eb.at[0], agg_sh.at[jb.at[0]], add=True)

    plsc.subcore_barrier()

    base = s * _ROWS_PER_TILE
    pltpu.sync_copy(agg_sh.at[pl.ds(base, _ROWS_PER_TILE)], wb)
    pltpu.sync_copy(wb, out_hbm.at[pl.ds(c * N + base, _ROWS_PER_TILE)])

  return k(e2, j2d)


def _node_mlp_tc(x, a0, a1, w1a, w1b, b1, w2, b2, w3, b3, ng, nbt):
  def body(x_ref, a0_ref, a1_ref, w1a_ref, w1b_ref, b1_ref, w2_ref, b2_ref,
           w3_ref, b3_ref, ng_ref, nbt_ref, out_ref):
    xv = x_ref[...]
    agg = a0_ref[...] + a1_ref[...]
    n1 = (jnp.dot(xv, w1a_ref[...], preferred_element_type=jnp.float32) +
          jnp.dot(agg, w1b_ref[...], preferred_element_type=jnp.float32) +
          b1_ref[...])
    h = jax.nn.silu(n1)
    h = jax.nn.silu(jnp.dot(h, w2_ref[...],
                            preferred_element_type=jnp.float32) + b2_ref[...])
    h3 = jnp.dot(h, w3_ref[...],
                 preferred_element_type=jnp.float32) + b3_ref[...]
    m = jnp.mean(h3, axis=-1, keepdims=True)
    cv = h3 - m
    var = jnp.mean(cv * cv, axis=-1, keepdims=True)
    out_ref[...] = xv + cv * lax.rsqrt(var + 1e-5) * ng_ref[...] + nbt_ref[...]

  bn = 2000
  grid = N // bn
  wspec = pl.BlockSpec((D, D), lambda i: (0, 0))
  bspec = pl.BlockSpec((1, D), lambda i: (0, 0))
  return pl.pallas_call(
      body,
      grid=(grid,),
      in_specs=[
          pl.BlockSpec((bn, D), lambda i: (i, 0)),
          pl.BlockSpec((bn, DE), lambda i: (i, 0)),
          pl.BlockSpec((bn, DE), lambda i: (i, 0)),
          pl.BlockSpec((D, D), lambda i: (0, 0)),
          pl.BlockSpec((DE, D), lambda i: (0, 0)),
          bspec, wspec, bspec, wspec, bspec, bspec, bspec,
      ],
      out_specs=pl.BlockSpec((bn, D), lambda i: (i, 0)),
      out_shape=jax.ShapeDtypeStruct((N, D), jnp.float32),
  )(x, a0, a1, w1a, w1b, b1, w2, b2, w3, b3, ng, nbt)


def kernel(x, edge_index, edge_attr, eW1, eb1, eW2, eb2, eW3, eb3, eg, ebt,
           nW1, nb1, nW2, nb2, nW3, nb3, ng, nbt):
  i = edge_index[0].astype(jnp.int32)
  j = edge_index[1].astype(jnp.int32)
  i2d = i.reshape(_NCHUNK, _CHUNK)
  j2d = j.reshape(_NCHUNK, _CHUNK)

  p, q = _pq_tc(x, eW1[:D], eW1[D:2 * D], eb1.reshape(1, DE))

  t = jnp.concatenate([p, q], axis=0)
  cidx = jnp.concatenate([i2d, j2d + N], axis=1)
  g3 = _gather_sc(t, cidx)

  eye8 = jnp.eye(8, dtype=jnp.float32)
  w1bd = jnp.kron(eye8, eW1[2 * D:])
  w2bd = jnp.kron(eye8, eW2)
  w3bd = jnp.kron(eye8, eW3)
  jbd = jnp.kron(eye8, jnp.full((DE, DE), 1.0 / DE, dtype=jnp.float32))
  b2t = jnp.tile(eb2, 8).reshape(1, D)
  b3t = jnp.tile(eb3, 8).reshape(1, D)
  egt = jnp.tile(eg, 8).reshape(1, D)
  ebtt = jnp.tile(ebt, 8).reshape(1, D)
  e2r, edge_attr2 = _edge_mlp_tc(g3, edge_attr,
                                 w1bd, w2bd, w3bd, b2t, b3t, egt, ebtt, jbd)

  aggp = _scatter_sc(e2r.reshape(_NCHUNK, _CHUNK, DE), j2d)

  x2 = _node_mlp_tc(x, aggp[:N], aggp[N:], nW1[:D], nW1[D:],
                    nb1.reshape(1, D), nW2, nb2.reshape(1, D),
                    nW3, nb3.reshape(1, D), ng.reshape(1, D),
                    nbt.reshape(1, D))
  return (x2, edge_attr2)


# speedup vs baseline: 1.1125x; 1.1125x over previous
import functools

import jax
import jax.numpy as jnp
from jax import lax
from jax.experimental import pallas as pl
from jax.experimental.pallas import tpu as pltpu
from jax.experimental.pallas import tpu_sc as plsc

N = 10000
E = 320000
D = 128
DE = 16

_CHUNK = 128
_NCHUNK = E // _CHUNK
_NW = 32
_MAXTRIP = (_NCHUNK + _NW - 1) // _NW
_ROWS_PER_TILE = N // 16


def _pq_tc(x, w1a, w1b, b1):
  def body(x_ref, wa_ref, wb_ref, b_ref, p_ref, q_ref):
    xv = x_ref[...]
    p_ref[...] = jnp.dot(xv, wa_ref[...],
                         preferred_element_type=jnp.float32) + b_ref[...]
    q_ref[...] = jnp.dot(xv, wb_ref[...], preferred_element_type=jnp.float32)

  bn = 2000
  grid = N // bn
  return pl.pallas_call(
      body,
      grid=(grid,),
      in_specs=[
          pl.BlockSpec((bn, D), lambda i: (i, 0)),
          pl.BlockSpec((D, DE), lambda i: (0, 0)),
          pl.BlockSpec((D, DE), lambda i: (0, 0)),
          pl.BlockSpec((1, DE), lambda i: (0, 0)),
      ],
      out_specs=[
          pl.BlockSpec((bn, DE), lambda i: (i, 0)),
          pl.BlockSpec((bn, DE), lambda i: (i, 0)),
      ],
      out_shape=[
          jax.ShapeDtypeStruct((N, DE), jnp.float32),
          jax.ShapeDtypeStruct((N, DE), jnp.float32),
      ],
  )(x, w1a, w1b, b1)


_KK = 13
_NBATCH = 6


def _gather_sc(t, cidx):
  mesh = plsc.VectorSubcoreMesh(core_axis_name="c", subcore_axis_name="s")

  @functools.partial(
      pl.kernel,
      mesh=mesh,
      compiler_params=pltpu.CompilerParams(use_tc_tiling_on_sc=False),
      out_type=jax.ShapeDtypeStruct((_NCHUNK, DE, _CHUNK), jnp.float32),
      scratch_types=[
          pltpu.VMEM((_KK, 2 * _CHUNK), jnp.int32),
          pltpu.VMEM((_KK, _CHUNK, DE), jnp.float32),
          pltpu.VMEM((_KK, _CHUNK, DE), jnp.float32),
          pltpu.VMEM((_KK, DE, _CHUNK), jnp.float32),
          pltpu.SemaphoreType.DMA,
          pltpu.SemaphoreType.DMA,
          pltpu.SemaphoreType.DMA,
      ],
      name="gather_sc",
  )
  def k(t_hbm, cidx_hbm, g_hbm, idxb, pb, qb, gb, s_i, s_g, s_o):
    c = lax.axis_index("c")
    s = lax.axis_index("s")
    wid = s * 2 + c

    def repack(kk):
      def rows(r8, carry):
        for m in range(8):
          v = pb[kk, r8 * 8 + m, :] + qb[kk, r8 * 8 + m, :]
          gb[kk, r8, m * DE:(m + 1) * DE] = v
        return carry

      lax.fori_loop(0, DE, rows, 0)

    def batch(g, carry):
      cps = []
      for kk in range(_KK):
        cid = wid + (g * _KK + kk) * _NW
        cps.append(pltpu.async_copy(cidx_hbm.at[cid], idxb.at[kk], s_i))
      for cp in cps:
        cp.wait()
      cps = []
      for kk in range(_KK):
        cps.append(pltpu.async_copy(
            t_hbm.at[idxb.at[kk, pl.ds(0, _CHUNK)]], pb.at[kk], s_g))
        cps.append(pltpu.async_copy(
            t_hbm.at[idxb.at[kk, pl.ds(_CHUNK, _CHUNK)]], qb.at[kk], s_g))
      for cp in cps:
        cp.wait()
      for kk in range(_KK):
        repack(kk)
      cps = []
      for kk in range(_KK):
        cid = wid + (g * _KK + kk) * _NW
        cps.append(pltpu.async_copy(gb.at[kk], g_hbm.at[cid], s_o))
      for cp in cps:
        cp.wait()
      return carry

    lax.fori_loop(0, _NBATCH, batch, 0)

    cid = _NBATCH * _KK * _NW + wid

    @pl.when(cid < _NCHUNK)
    def _():
      pltpu.sync_copy(cidx_hbm.at[cid], idxb.at[0])
      cp1 = pltpu.async_copy(
          t_hbm.at[idxb.at[0, pl.ds(0, _CHUNK)]], pb.at[0], s_g)
      cp2 = pltpu.async_copy(
          t_hbm.at[idxb.at[0, pl.ds(_CHUNK, _CHUNK)]], qb.at[0], s_g)
      cp1.wait()
      cp2.wait()
      repack(0)
      pltpu.sync_copy(gb.at[0], g_hbm.at[cid])

  return k(t, cidx)


def _edge_mlp_tc(g3, ea, w1bd, w2bd, w3bd, b2t, b3t, egt, ebtt, jbd):
  bn = 4000
  rows = E // 8

  def body(g_ref, ea_ref, w1_ref, w2_ref, w3_ref, b2_ref, b3_ref,
           eg_ref, ebt_ref, jm_ref, out_ref):
    eav = ea_ref[...]
    gv = g_ref[...].reshape(bn, _CHUNK)
    h1 = gv + jnp.dot(eav, w1_ref[...], preferred_element_type=jnp.float32)
    h1 = jax.nn.silu(h1)
    h2 = jax.nn.silu(jnp.dot(h1, w2_ref[...],
                             preferred_element_type=jnp.float32) + b2_ref[...])
    h3 = jnp.dot(h2, w3_ref[...],
                 preferred_element_type=jnp.float32) + b3_ref[...]
    jm = jm_ref[...]
    m_ = jnp.dot(h3, jm, preferred_element_type=jnp.float32)
    msq = jnp.dot(h3 * h3, jm, preferred_element_type=jnp.float32)
    var = msq - m_ * m_
    ln = (h3 - m_) * lax.rsqrt(var + 1e-5) * eg_ref[...] + ebt_ref[...]
    out_ref[...] = eav + ln

  grid = rows // bn
  wspec = pl.BlockSpec((D, D), lambda i: (0, 0))
  bspec = pl.BlockSpec((1, D), lambda i: (0, 0))
  return pl.pallas_call(
      body,
      grid=(grid,),
      in_specs=[
          pl.BlockSpec((bn // DE, DE, _CHUNK), lambda i: (i, 0, 0)),
          pl.BlockSpec((bn, D), lambda i: (i, 0)),
          wspec, wspec, wspec, bspec, bspec, bspec, bspec, wspec,
      ],
      out_specs=pl.BlockSpec((bn, D), lambda i: (i, 0)),
      out_shape=jax.ShapeDtypeStruct((rows, D), jnp.float32),
  )(g3, ea, w1bd, w2bd, w3bd, b2t, b3t, egt, ebtt, jbd)


def _scatter_sc(e2, j2d):
  mesh = plsc.VectorSubcoreMesh(core_axis_name="c", subcore_axis_name="s")

  @functools.partial(
      pl.kernel,
      mesh=mesh,
      compiler_params=pltpu.CompilerParams(use_tc_tiling_on_sc=False),
      out_type=jax.ShapeDtypeStruct((2 * N, DE), jnp.float32),
      name="scatter_sc",
      scratch_types=[
          pltpu.VMEM((_KK, _CHUNK), jnp.int32),
          pltpu.VMEM((_KK, _CHUNK, DE), jnp.float32),
          pltpu.VMEM((_ROWS_PER_TILE, DE), jnp.float32),
          pltpu.VMEM_SHARED((N, DE), jnp.float32),
          pltpu.SemaphoreType.DMA,
          pltpu.SemaphoreType.DMA,
      ],
  )
  def k(e2_hbm, j_hbm, out_hbm, jb, eb, wb, agg_sh, s_i, s_a):
    c = lax.axis_index("c")
    s = lax.axis_index("s")
    wid = s * 2 + c

    def zbody(r, carry):
      wb[r, :] = jnp.zeros((DE,), jnp.float32)
      return carry

    lax.fori_loop(0, _ROWS_PER_TILE, zbody, 0)
    pltpu.sync_copy(wb, agg_sh.at[pl.ds(s * _ROWS_PER_TILE, _ROWS_PER_TILE)])
    plsc.subcore_barrier()

    def batch(g, carry):
      cps = []
      for kk in range(_KK):
        cid = wid + (g * _KK + kk) * _NW
        cps.append(pltpu.async_copy(j_hbm.at[cid], jb.at[kk], s_i))
        cps.append(pltpu.async_copy(e2_hbm.at[cid], eb.at[kk], s_i))
      for cp in cps:
        cp.wait()
      cps = []
      for kk in range(_KK):
        cps.append(pltpu.async_copy(
            eb.at[kk], agg_sh.at[jb.at[kk]], s_a, add=True))
      for cp in cps:
        cp.wait()
      return carry

    lax.fori_loop(0, _NBATCH, batch, 0)

    cid = _NBATCH * _KK * _NW + wid

    @pl.when(cid < _NCHUNK)
    def _():
      pltpu.sync_copy(j_hbm.at[cid], jb.at[0])
      pltpu.sync_copy(e2_hbm.at[cid], eb.at[0])
      pltpu.sync_copy(eb.at[0], agg_sh.at[jb.at[0]], add=True)

    plsc.subcore_barrier()

    base = s * _ROWS_PER_TILE
    pltpu.sync_copy(agg_sh.at[pl.ds(base, _ROWS_PER_TILE)], wb)
    pltpu.sync_copy(wb, out_hbm.at[pl.ds(c * N + base, _ROWS_PER_TILE)])

  return k(e2, j2d)


def _node_mlp_tc(x, aggp, w1a, w1b, b1, w2, b2, w3, b3, ng, nbt):
  def body(x_ref, a0_ref, a1_ref, w1a_ref, w1b_ref, b1_ref, w2_ref, b2_ref,
           w3_ref, b3_ref, ng_ref, nbt_ref, out_ref):
    xv = x_ref[...]
    agg = a0_ref[...] + a1_ref[...]
    n1 = (jnp.dot(xv, w1a_ref[...], preferred_element_type=jnp.float32) +
          jnp.dot(agg, w1b_ref[...], preferred_element_type=jnp.float32) +
          b1_ref[...])
    h = jax.nn.silu(n1)
    h = jax.nn.silu(jnp.dot(h, w2_ref[...],
                            preferred_element_type=jnp.float32) + b2_ref[...])
    h3 = jnp.dot(h, w3_ref[...],
                 preferred_element_type=jnp.float32) + b3_ref[...]
    m = jnp.mean(h3, axis=-1, keepdims=True)
    cv = h3 - m
    var = jnp.mean(cv * cv, axis=-1, keepdims=True)
    out_ref[...] = xv + cv * lax.rsqrt(var + 1e-5) * ng_ref[...] + nbt_ref[...]

  bn = 2000
  grid = N // bn
  wspec = pl.BlockSpec((D, D), lambda i: (0, 0))
  bspec = pl.BlockSpec((1, D), lambda i: (0, 0))
  return pl.pallas_call(
      body,
      grid=(grid,),
      in_specs=[
          pl.BlockSpec((bn, D), lambda i: (i, 0)),
          pl.BlockSpec((bn, DE), lambda i: (i, 0)),
          pl.BlockSpec((bn, DE), lambda i: (N // bn + i, 0)),
          pl.BlockSpec((D, D), lambda i: (0, 0)),
          pl.BlockSpec((DE, D), lambda i: (0, 0)),
          bspec, wspec, bspec, wspec, bspec, bspec, bspec,
      ],
      out_specs=pl.BlockSpec((bn, D), lambda i: (i, 0)),
      out_shape=jax.ShapeDtypeStruct((N, D), jnp.float32),
  )(x, aggp, aggp, w1a, w1b, b1, w2, b2, w3, b3, ng, nbt)


def kernel(x, edge_index, edge_attr, eW1, eb1, eW2, eb2, eW3, eb3, eg, ebt,
           nW1, nb1, nW2, nb2, nW3, nb3, ng, nbt):
  i = edge_index[0].astype(jnp.int32)
  j = edge_index[1].astype(jnp.int32)
  i2d = i.reshape(_NCHUNK, _CHUNK)
  j2d = j.reshape(_NCHUNK, _CHUNK)

  p, q = _pq_tc(x, eW1[:D], eW1[D:2 * D], eb1.reshape(1, DE))

  t = jnp.concatenate([p, q], axis=0)
  cidx = jnp.concatenate([i2d, j2d + N], axis=1)
  g3 = _gather_sc(t, cidx)

  eye8 = jnp.eye(8, dtype=jnp.float32)
  w1bd = jnp.kron(eye8, eW1[2 * D:])
  w2bd = jnp.kron(eye8, eW2)
  w3bd = jnp.kron(eye8, eW3)
  jbd = jnp.kron(eye8, jnp.full((DE, DE), 1.0 / DE, dtype=jnp.float32))
  b2t = jnp.tile(eb2, 8).reshape(1, D)
  b3t = jnp.tile(eb3, 8).reshape(1, D)
  egt = jnp.tile(eg, 8).reshape(1, D)
  ebtt = jnp.tile(ebt, 8).reshape(1, D)
  e2r = _edge_mlp_tc(g3, edge_attr.reshape(E // 8, D),
                     w1bd, w2bd, w3bd, b2t, b3t, egt, ebtt, jbd)
  edge_attr2 = e2r.reshape(E, DE)

  aggp = _scatter_sc(e2r.reshape(_NCHUNK, _CHUNK, DE), j2d)

  x2 = _node_mlp_tc(x, aggp, nW1[:D], nW1[D:],
                    nb1.reshape(1, D), nW2, nb2.reshape(1, D),
                    nW3, nb3.reshape(1, D), ng.reshape(1, D),
                    nbt.reshape(1, D))
  return (x2, edge_attr2)
